# baseline (speedup 1.0000x reference)
import jax
import jax.numpy as jnp
from jax import lax
from jax.experimental import pallas as pl
from jax.experimental.pallas import tpu as pltpu
from jax.experimental.pallas import tpu_sc as plsc

N_NODES = 10000
D = 128
A_DIM = 25
E_EDGES = 320000

NC = 2
NS = 16
L = 16
CHUNK = 128
CPT = 160
HCPT = CPT // NC
EP = NS * CPT * CHUNK
OWN = 5120
SH_ROWS = 5248
ROWS_PER_TILE = SH_ROWS // NS
HIST = 10240
DUMMY = N_NODES
CAP = HCPT * CHUNK
CAPL = CAP + L
NEL = 2 * CAP + 2 * CHUNK


def _mesh():
    return plsc.VectorSubcoreMesh(core_axis_name="c", subcore_axis_name="s")


def _route_body(src_hbm, dst_hbm, hist_out, rsrc_out, rdst_out, cnts_out,
                src_v, dst_v, l0s, l0d, l1s, l1d, hist_v, cvec_v):
    c = lax.axis_index("c")
    s = lax.axis_index("s")
    pltpu.sync_copy(src_hbm.at[s, pl.ds(c * HCPT, HCPT)], src_v)
    pltpu.sync_copy(dst_hbm.at[s, pl.ds(c * HCPT, HCPT)], dst_v)

    @pl.loop(0, HIST // L)
    def _(j):
        hist_v[pl.ds(j * L, L)] = jnp.zeros((L,), jnp.float32)

    ones = jnp.ones((L,), jnp.float32)

    @pl.loop(0, HCPT, init_carry=(jnp.int32(0), jnp.int32(0)))
    def offs(j, carry):
        off0, off1 = carry
        for g in range(CHUNK // L):
            sv = src_v[j, pl.ds(g * L, L)]
            dv = dst_v[j, pl.ds(g * L, L)]
            plsc.addupdate_scatter(hist_v, [dv], ones)
            m0 = dv < OWN
            m1 = jnp.logical_not(m0)
            n0 = jnp.max(plsc.all_reduce_population_count(m0))
            plsc.store_compressed(l0s.at[pl.ds(off0, L)], sv, mask=m0)
            plsc.store_compressed(l0d.at[pl.ds(off0, L)], dv, mask=m0)
            plsc.store_compressed(l1s.at[pl.ds(off1, L)], sv, mask=m1)
            plsc.store_compressed(l1d.at[pl.ds(off1, L)], dv, mask=m1)
            off0 = off0 + n0
            off1 = off1 + (L - n0)
        return off0, off1

    off0, off1 = offs
    pltpu.sync_copy(hist_v, hist_out.at[c * NS + s])
    pltpu.sync_copy(l0s.at[pl.ds(0, CAP)], rsrc_out.at[s, c, 0])
    pltpu.sync_copy(l0d.at[pl.ds(0, CAP)], rdst_out.at[s, c, 0])
    pltpu.sync_copy(l1s.at[pl.ds(0, CAP)], rsrc_out.at[s, c, 1])
    pltpu.sync_copy(l1d.at[pl.ds(0, CAP)], rdst_out.at[s, c, 1])
    cvec_v[0, pl.ds(0, L)] = jnp.full((L,), off0, jnp.int32)
    cvec_v[1, pl.ds(0, L)] = jnp.full((L,), off1, jnp.int32)
    pltpu.sync_copy(cvec_v.at[0], cnts_out.at[s * 4 + c * 2])
    pltpu.sync_copy(cvec_v.at[1], cnts_out.at[s * 4 + c * 2 + 1])


def _route_kernel(src_idx, dst_idx):
    return pl.kernel(
        _route_body,
        out_type=(jax.ShapeDtypeStruct((NC * NS, HIST), jnp.float32),
                  jax.ShapeDtypeStruct((NS, NC, NC, CAP), jnp.int32),
                  jax.ShapeDtypeStruct((NS, NC, NC, CAP), jnp.int32),
                  jax.ShapeDtypeStruct((NS * 4, L), jnp.int32)),
        mesh=_mesh(),
        scratch_types=[
            pltpu.VMEM((HCPT, CHUNK), jnp.int32),
            pltpu.VMEM((HCPT, CHUNK), jnp.int32),
            pltpu.VMEM((CAPL,), jnp.int32),
            pltpu.VMEM((CAPL,), jnp.int32),
            pltpu.VMEM((CAPL,), jnp.int32),
            pltpu.VMEM((CAPL,), jnp.int32),
            pltpu.VMEM((HIST,), jnp.float32),
            pltpu.VMEM((2, L), jnp.int32),
        ],
        compiler_params=pltpu.CompilerParams(needs_layout_passes=False),
    )(src_idx, dst_idx)


NBUF = 2


def _agg_body(y_hbm, rsrc_hbm, rdst_hbm, cnts_hbm, zeros_hbm, out_hbm,
              vsrc, vdst, cts_v, sidx_v, lidx_v, rows_v, agg_sh,
              gsems, ssems):
    c = lax.axis_index("c")
    s = lax.axis_index("s")
    base = c * OWN
    pltpu.sync_copy(rsrc_hbm.at[s, 0, c], vsrc.at[pl.ds(0, CAP)])
    pltpu.sync_copy(rdst_hbm.at[s, 0, c], vdst.at[pl.ds(0, CAP)])
    pltpu.sync_copy(rsrc_hbm.at[s, 1, c], vsrc.at[pl.ds(CAP, CAP)])
    pltpu.sync_copy(rdst_hbm.at[s, 1, c], vdst.at[pl.ds(CAP, CAP)])
    pltpu.sync_copy(cnts_hbm.at[s * 4 + c], cts_v.at[0])
    pltpu.sync_copy(cnts_hbm.at[s * 4 + 2 + c], cts_v.at[1])
    pltpu.sync_copy(zeros_hbm, agg_sh.at[pl.ds(s * ROWS_PER_TILE, CHUNK)])
    pltpu.sync_copy(zeros_hbm,
                    agg_sh.at[pl.ds(s * ROWS_PER_TILE + CHUNK, CHUNK)])
    pltpu.sync_copy(zeros_hbm.at[pl.ds(0, ROWS_PER_TILE - 2 * CHUNK)],
                    agg_sh.at[pl.ds(s * ROWS_PER_TILE + 2 * CHUNK,
                                    ROWS_PER_TILE - 2 * CHUNK)])
    plsc.subcore_barrier()

    cnt0 = jnp.max(cts_v[0, pl.ds(0, L)])
    cnt1 = jnp.max(cts_v[1, pl.ds(0, L)])
    nch0 = (cnt0 + CHUNK - 1) // CHUNK
    nch1 = (cnt1 + CHUNK - 1) // CHUNK
    nch2 = jnp.maximum(((nch0 + nch1 + 1) // 2) * 2, 2)

    def prep(q, b):
        in0 = q < nch0
        fb = jnp.where(in0, q * CHUNK, (q - nch0) * CHUNK + CAP)
        pb = jnp.where(in0, q * CHUNK, (q - nch0) * CHUNK)
        cnt_here = jnp.where(in0, cnt0, cnt1)
        cnt_vec = jnp.full((L,), cnt_here, jnp.int32)
        for g in range(CHUNK // L):
            sv = vsrc[pl.ds(fb + g * L, L)]
            dv = vdst[pl.ds(fb + g * L, L)]
            pos = jnp.full((L,), pb + g * L, jnp.int32) + \
                jnp.arange(L, dtype=jnp.int32)
            ok = pos < cnt_vec
            dummy = jnp.arange(L, dtype=jnp.int32) + (OWN + g * L)
            sidx_v[b, pl.ds(g * L, L)] = jnp.where(ok, sv, 0)
            lidx_v[b, pl.ds(g * L, L)] = jnp.where(ok, dv - base, dummy)

    HC = CHUNK // 2

    def g_start(b):
        pltpu.async_copy(y_hbm.at[sidx_v.at[b, pl.ds(0, HC)]],
                         rows_v.at[b, pl.ds(0, HC)], gsems.at[b, 0])
        pltpu.async_copy(y_hbm.at[sidx_v.at[b, pl.ds(HC, HC)]],
                         rows_v.at[b, pl.ds(HC, HC)], gsems.at[b, 1])

    def g_wait(b):
        pltpu.make_async_copy(y_hbm.at[sidx_v.at[b, pl.ds(0, HC)]],
                              rows_v.at[b, pl.ds(0, HC)], gsems.at[b, 0]).wait()
        pltpu.make_async_copy(y_hbm.at[sidx_v.at[b, pl.ds(HC, HC)]],
                              rows_v.at[b, pl.ds(HC, HC)], gsems.at[b, 1]).wait()

    for b in range(NBUF):
        prep(jnp.int32(b), b)
        g_start(b)

    @pl.loop(0, nch2, step=NBUF)
    def _(j):
        for b in range(NBUF):
            g_wait(b)
        for b in range(NBUF):
            pltpu.async_copy(rows_v.at[b], agg_sh.at[lidx_v.at[b]],
                             ssems.at[b], add=True)
        for b in range(NBUF):
            @pl.when(j + b + NBUF < nch2)
            def _():
                pltpu.make_async_copy(rows_v.at[b],
                                      agg_sh.at[lidx_v.at[b]],
                                      ssems.at[b]).wait()
                prep(j + b + NBUF, b)
                g_start(b)

    for b in range(NBUF):
        pltpu.make_async_copy(rows_v.at[b], agg_sh.at[lidx_v.at[b]],
                              ssems.at[b]).wait()
    plsc.subcore_barrier()
    pltpu.sync_copy(agg_sh.at[pl.ds(s * ROWS_PER_TILE, ROWS_PER_TILE)],
                    out_hbm.at[c, pl.ds(s * ROWS_PER_TILE, ROWS_PER_TILE)])


def _agg_kernel(y, rsrc, rdst, cnts, zeros128):
    return pl.kernel(
        _agg_body,
        out_type=jax.ShapeDtypeStruct((NC, SH_ROWS, D), jnp.float32),
        mesh=_mesh(),
        scratch_types=[
            pltpu.VMEM((NEL,), jnp.int32),
            pltpu.VMEM((NEL,), jnp.int32),
            pltpu.VMEM((2, L), jnp.int32),
            pltpu.VMEM((NBUF, CHUNK), jnp.int32),
            pltpu.VMEM((NBUF, CHUNK), jnp.int32),
            pltpu.VMEM((NBUF, CHUNK, D), jnp.float32),
            pltpu.VMEM_SHARED((SH_ROWS, D), jnp.float32),
            pltpu.SemaphoreType.DMA((NBUF, 2)),
            pltpu.SemaphoreType.DMA((NBUF,)),
        ],
        compiler_params=pltpu.CompilerParams(needs_layout_passes=False),
    )(y, rsrc, rdst, cnts, zeros128)


def _tc1_body(x_ref, w1_ref, deg_ref, y_ref, dinv_ref):
    deg = jnp.sum(deg_ref[...], axis=0)[:N_NODES] + 1.0
    dinv = lax.rsqrt(deg)
    xw = jnp.dot(x_ref[...], w1_ref[...], preferred_element_type=jnp.float32)
    y_ref[...] = xw * dinv[:, None]
    dinv_ref[...] = dinv[:, None]


def _agg_full(agg_ref):
    return jnp.concatenate([agg_ref[0, :OWN], agg_ref[1, :N_NODES - OWN]],
                           axis=0)


def _tc2_body(y1_ref, agg_ref, dinv_ref, b1_ref, w2_ref, y2_ref):
    f = _agg_full(agg_ref) + y1_ref[...]
    h = jax.nn.relu(f * dinv_ref[...] + b1_ref[...])
    y2_ref[...] = jnp.dot(h, w2_ref[...],
                          preferred_element_type=jnp.float32) * dinv_ref[...]


def _tc3_body(y2_ref, agg_ref, dinv_ref, b2_ref, gamma_ref, beta_ref,
              wa_ref, ba_ref, wq_ref, bq_ref,
              logits_ref, sv_ref, h_ref):
    f = _agg_full(agg_ref) + y2_ref[...]
    h = jax.nn.relu(f * dinv_ref[...] + b2_ref[...])
    mu = jnp.mean(h, axis=-1, keepdims=True)
    var = jnp.mean((h - mu) * (h - mu), axis=-1, keepdims=True)
    ln = (h - mu) / jnp.sqrt(var + 1e-5) * gamma_ref[...] + beta_ref[...]
    h_ref[...] = ln
    logits_ref[...] = jnp.dot(ln, wa_ref[...],
                              preferred_element_type=jnp.float32) + ba_ref[...]
    q = jnp.dot(ln, wq_ref[...],
                preferred_element_type=jnp.float32) + bq_ref[...]
    sv_ref[...] = jnp.mean(q, axis=-1, keepdims=True)


def _tc1(x2d, W1, degp):
    return pl.pallas_call(
        _tc1_body,
        out_shape=(jax.ShapeDtypeStruct((N_NODES, D), jnp.float32),
                   jax.ShapeDtypeStruct((N_NODES, 1), jnp.float32)),
    )(x2d, W1, degp)


def _tc2(y1, agg, dinv, b1, W2):
    return pl.pallas_call(
        _tc2_body,
        out_shape=jax.ShapeDtypeStruct((N_NODES, D), jnp.float32),
    )(y1, agg, dinv, b1, W2)


def _tc3(y2, agg, dinv, b2, gamma, beta, Wa, ba, Wq, bq):
    return pl.pallas_call(
        _tc3_body,
        out_shape=(jax.ShapeDtypeStruct((N_NODES, A_DIM), jnp.float32),
                   jax.ShapeDtypeStruct((N_NODES, 1), jnp.float32),
                   jax.ShapeDtypeStruct((N_NODES, D), jnp.float32)),
    )(y2, agg, dinv, b2, gamma, beta, Wa, ba, Wq, bq)


@jax.jit
def kernel(x, edge_index, W1, b1, W2, b2, gamma, beta, Wa, ba, Wq, bq):
    batch, seq, _ = x.shape
    x2d = x.reshape(batch * seq, D)

    e = edge_index.astype(jnp.int32)
    pad = EP - E_EDGES
    src = jnp.concatenate([e[0], jnp.zeros((pad,), jnp.int32)])
    dst = jnp.concatenate([e[1], jnp.full((pad,), DUMMY, jnp.int32)])
    src = src.reshape(NS, CPT, CHUNK)
    dst = dst.reshape(NS, CPT, CHUNK)

    zeros128 = jnp.zeros((CHUNK, D), jnp.float32)

    degp, rsrc, rdst, cnts = _route_kernel(src, dst)
    y1, dinv = _tc1(x2d, W1, degp)
    agg1 = _agg_kernel(y1, rsrc, rdst, cnts, zeros128)
    y2 = _tc2(y1, agg1, dinv, b1.reshape(1, D), W2)
    agg2 = _agg_kernel(y2, rsrc, rdst, cnts, zeros128)
    logits, sv, h = _tc3(y2, agg2, dinv, b2.reshape(1, D),
                         gamma.reshape(1, D), beta.reshape(1, D),
                         Wa, ba.reshape(1, A_DIM), Wq, bq.reshape(1, A_DIM))
    return (logits, sv, h)


# speedup vs baseline: 1.9744x; 1.9744x over previous
import jax
import jax.numpy as jnp
from jax import lax
from jax.experimental import pallas as pl
from jax.experimental.pallas import tpu as pltpu
from jax.experimental.pallas import tpu_sc as plsc

N_NODES = 10000
D = 128
A_DIM = 25
E_EDGES = 320000

NC = 2
NS = 16
L = 16
CHUNK = 128
CPT = 160
HCPT = CPT // NC
EP = NS * CPT * CHUNK
OWN = 5120
SH_ROWS = 5248
ROWS_PER_TILE = SH_ROWS // NS
HIST = 10240
DUMMY = N_NODES
CAP = HCPT * CHUNK
CAPL = CAP + L
NEL = 2 * CAP + 2 * CHUNK


def _mesh():
    return plsc.VectorSubcoreMesh(core_axis_name="c", subcore_axis_name="s")


def _route_body(src_hbm, dst_hbm, hist_out, rsrc_out, rdst_out, cnts_out,
                src_v, dst_v, l0s, l0d, l1s, l1d, hist_v, cvec_v):
    c = lax.axis_index("c")
    s = lax.axis_index("s")
    pltpu.sync_copy(src_hbm.at[s, pl.ds(c * HCPT, HCPT)], src_v)
    pltpu.sync_copy(dst_hbm.at[s, pl.ds(c * HCPT, HCPT)], dst_v)

    @pl.loop(0, HIST // L)
    def _(j):
        hist_v[pl.ds(j * L, L)] = jnp.zeros((L,), jnp.float32)

    ones = jnp.ones((L,), jnp.float32)

    @pl.loop(0, HCPT, init_carry=(jnp.int32(0), jnp.int32(0)))
    def offs(j, carry):
        off0, off1 = carry
        for g in range(CHUNK // L):
            sv = src_v[j, pl.ds(g * L, L)]
            dv = dst_v[j, pl.ds(g * L, L)]
            plsc.addupdate_scatter(hist_v, [dv], ones)
            m0 = dv < OWN
            m1 = jnp.logical_not(m0) & (dv < DUMMY)
            n0 = jnp.max(plsc.all_reduce_population_count(m0))
            plsc.store_compressed(l0s.at[pl.ds(off0, L)], sv, mask=m0)
            plsc.store_compressed(l0d.at[pl.ds(off0, L)], dv, mask=m0)
            plsc.store_compressed(l1s.at[pl.ds(off1, L)], sv, mask=m1)
            plsc.store_compressed(l1d.at[pl.ds(off1, L)], dv, mask=m1)
            off0 = off0 + n0
            n1 = jnp.max(plsc.all_reduce_population_count(m1))
            off1 = off1 + n1
        return off0, off1

    off0, off1 = offs
    pltpu.sync_copy(hist_v, hist_out.at[c * NS + s])
    pltpu.sync_copy(l0s.at[pl.ds(0, CAP)], rsrc_out.at[s, c, 0])
    pltpu.sync_copy(l0d.at[pl.ds(0, CAP)], rdst_out.at[s, c, 0])
    pltpu.sync_copy(l1s.at[pl.ds(0, CAP)], rsrc_out.at[s, c, 1])
    pltpu.sync_copy(l1d.at[pl.ds(0, CAP)], rdst_out.at[s, c, 1])
    cvec_v[0, pl.ds(0, L)] = jnp.full((L,), off0, jnp.int32)
    cvec_v[1, pl.ds(0, L)] = jnp.full((L,), off1, jnp.int32)
    pltpu.sync_copy(cvec_v.at[0], cnts_out.at[s * 4 + c * 2])
    pltpu.sync_copy(cvec_v.at[1], cnts_out.at[s * 4 + c * 2 + 1])


def _route_kernel(src_idx, dst_idx):
    return pl.kernel(
        _route_body,
        out_type=(jax.ShapeDtypeStruct((NC * NS, HIST), jnp.float32),
                  jax.ShapeDtypeStruct((NS, NC, NC, CAP), jnp.int32),
                  jax.ShapeDtypeStruct((NS, NC, NC, CAP), jnp.int32),
                  jax.ShapeDtypeStruct((NS * 4, L), jnp.int32)),
        mesh=_mesh(),
        scratch_types=[
            pltpu.VMEM((HCPT, CHUNK), jnp.int32),
            pltpu.VMEM((HCPT, CHUNK), jnp.int32),
            pltpu.VMEM((CAPL,), jnp.int32),
            pltpu.VMEM((CAPL,), jnp.int32),
            pltpu.VMEM((CAPL,), jnp.int32),
            pltpu.VMEM((CAPL,), jnp.int32),
            pltpu.VMEM((HIST,), jnp.float32),
            pltpu.VMEM((2, L), jnp.int32),
        ],
        compiler_params=pltpu.CompilerParams(needs_layout_passes=False),
    )(src_idx, dst_idx)


NBUF = 2


def _agg_body(y_hbm, rsrc_hbm, rdst_hbm, cnts_hbm, zeros_hbm, out_hbm,
              vsrc, vdst, cts_v, sidx_v, lidx_v, rows_v, agg_sh,
              gsems, ssems):
    c = lax.axis_index("c")
    s = lax.axis_index("s")
    base = c * OWN
    pltpu.sync_copy(rsrc_hbm.at[s, 0, c], vsrc.at[pl.ds(0, CAP)])
    pltpu.sync_copy(rdst_hbm.at[s, 0, c], vdst.at[pl.ds(0, CAP)])
    pltpu.sync_copy(rsrc_hbm.at[s, 1, c], vsrc.at[pl.ds(CAP, CAP)])
    pltpu.sync_copy(rdst_hbm.at[s, 1, c], vdst.at[pl.ds(CAP, CAP)])
    pltpu.sync_copy(cnts_hbm.at[s * 4 + c], cts_v.at[0])
    pltpu.sync_copy(cnts_hbm.at[s * 4 + 2 + c], cts_v.at[1])
    pltpu.sync_copy(zeros_hbm, agg_sh.at[pl.ds(s * ROWS_PER_TILE, CHUNK)])
    pltpu.sync_copy(zeros_hbm,
                    agg_sh.at[pl.ds(s * ROWS_PER_TILE + CHUNK, CHUNK)])
    pltpu.sync_copy(zeros_hbm.at[pl.ds(0, ROWS_PER_TILE - 2 * CHUNK)],
                    agg_sh.at[pl.ds(s * ROWS_PER_TILE + 2 * CHUNK,
                                    ROWS_PER_TILE - 2 * CHUNK)])
    plsc.subcore_barrier()

    cnt0 = jnp.max(cts_v[0, pl.ds(0, L)])
    cnt1 = jnp.max(cts_v[1, pl.ds(0, L)])
    nch0 = (cnt0 + CHUNK - 1) // CHUNK
    nch1 = (cnt1 + CHUNK - 1) // CHUNK
    nch2 = jnp.maximum(((nch0 + nch1 + 1) // 2) * 2, 2)

    def prep(q, b):
        in0 = q < nch0
        fb = jnp.where(in0, q * CHUNK, (q - nch0) * CHUNK + CAP)
        pb = jnp.where(in0, q * CHUNK, (q - nch0) * CHUNK)
        cnt_here = jnp.where(in0, cnt0, cnt1)
        cnt_vec = jnp.full((L,), cnt_here, jnp.int32)
        for g in range(CHUNK // L):
            sv = vsrc[pl.ds(fb + g * L, L)]
            dv = vdst[pl.ds(fb + g * L, L)]
            pos = jnp.full((L,), pb + g * L, jnp.int32) + \
                jnp.arange(L, dtype=jnp.int32)
            ok = pos < cnt_vec
            dummy = jnp.arange(L, dtype=jnp.int32) + (OWN + g * L)
            sidx_v[b, pl.ds(g * L, L)] = jnp.where(ok, sv, 0)
            lidx_v[b, pl.ds(g * L, L)] = jnp.where(ok, dv - base, dummy)

    def g_start(b):
        pltpu.async_copy(y_hbm.at[sidx_v.at[b]], rows_v.at[b], gsems.at[b])

    for b in range(NBUF):
        prep(jnp.int32(b), b)
        g_start(b)

    @pl.loop(0, nch2, step=NBUF)
    def _(j):
        for b in range(NBUF):
            jj = j + b
            pltpu.make_async_copy(y_hbm.at[sidx_v.at[b]],
                                  rows_v.at[b], gsems.at[b]).wait()
            pltpu.async_copy(rows_v.at[b], agg_sh.at[lidx_v.at[b]],
                             ssems.at[b], add=True)

            @pl.when(jj + NBUF < nch2)
            def _():
                pltpu.make_async_copy(rows_v.at[b],
                                      agg_sh.at[lidx_v.at[b]],
                                      ssems.at[b]).wait()
                prep(jj + NBUF, b)
                g_start(b)

    for b in range(NBUF):
        pltpu.make_async_copy(rows_v.at[b], agg_sh.at[lidx_v.at[b]],
                              ssems.at[b]).wait()
    plsc.subcore_barrier()
    pltpu.sync_copy(agg_sh.at[pl.ds(s * ROWS_PER_TILE, ROWS_PER_TILE)],
                    out_hbm.at[c, pl.ds(s * ROWS_PER_TILE, ROWS_PER_TILE)])


def _agg_kernel(y, rsrc, rdst, cnts, zeros128):
    return pl.kernel(
        _agg_body,
        out_type=jax.ShapeDtypeStruct((NC, SH_ROWS, D), jnp.float32),
        mesh=_mesh(),
        scratch_types=[
            pltpu.VMEM((NEL,), jnp.int32),
            pltpu.VMEM((NEL,), jnp.int32),
            pltpu.VMEM((2, L), jnp.int32),
            pltpu.VMEM((NBUF, CHUNK), jnp.int32),
            pltpu.VMEM((NBUF, CHUNK), jnp.int32),
            pltpu.VMEM((NBUF, CHUNK, D), jnp.float32),
            pltpu.VMEM_SHARED((SH_ROWS, D), jnp.float32),
            pltpu.SemaphoreType.DMA((NBUF,)),
            pltpu.SemaphoreType.DMA((NBUF,)),
        ],
        compiler_params=pltpu.CompilerParams(needs_layout_passes=False),
    )(y, rsrc, rdst, cnts, zeros128)


def _tc1_body(x_ref, w1_ref, deg_ref, y_ref, dinv_ref):
    deg = jnp.sum(deg_ref[...], axis=0)[:N_NODES] + 1.0
    dinv = lax.rsqrt(deg)
    xw = jnp.dot(x_ref[...], w1_ref[...], preferred_element_type=jnp.float32)
    y_ref[...] = xw * dinv[:, None]
    dinv_ref[...] = dinv[:, None]


def _agg_full(agg_ref):
    return jnp.concatenate([agg_ref[0, :OWN], agg_ref[1, :N_NODES - OWN]],
                           axis=0)


def _tc2_body(y1_ref, agg_ref, dinv_ref, b1_ref, w2_ref, y2_ref):
    f = _agg_full(agg_ref) + y1_ref[...]
    h = jax.nn.relu(f * dinv_ref[...] + b1_ref[...])
    y2_ref[...] = jnp.dot(h, w2_ref[...],
                          preferred_element_type=jnp.float32) * dinv_ref[...]


def _tc3_body(y2_ref, agg_ref, dinv_ref, b2_ref, gamma_ref, beta_ref,
              wa_ref, ba_ref, wq_ref, bq_ref,
              logits_ref, sv_ref, h_ref):
    f = _agg_full(agg_ref) + y2_ref[...]
    h = jax.nn.relu(f * dinv_ref[...] + b2_ref[...])
    mu = jnp.mean(h, axis=-1, keepdims=True)
    var = jnp.mean((h - mu) * (h - mu), axis=-1, keepdims=True)
    ln = (h - mu) / jnp.sqrt(var + 1e-5) * gamma_ref[...] + beta_ref[...]
    h_ref[...] = ln
    logits_ref[...] = jnp.dot(ln, wa_ref[...],
                              preferred_element_type=jnp.float32) + ba_ref[...]
    q = jnp.dot(ln, wq_ref[...],
                preferred_element_type=jnp.float32) + bq_ref[...]
    sv_ref[...] = jnp.mean(q, axis=-1, keepdims=True)


def _tc1(x2d, W1, degp):
    return pl.pallas_call(
        _tc1_body,
        out_shape=(jax.ShapeDtypeStruct((N_NODES, D), jnp.float32),
                   jax.ShapeDtypeStruct((N_NODES, 1), jnp.float32)),
    )(x2d, W1, degp)


def _tc2(y1, agg, dinv, b1, W2):
    return pl.pallas_call(
        _tc2_body,
        out_shape=jax.ShapeDtypeStruct((N_NODES, D), jnp.float32),
    )(y1, agg, dinv, b1, W2)


def _tc3(y2, agg, dinv, b2, gamma, beta, Wa, ba, Wq, bq):
    return pl.pallas_call(
        _tc3_body,
        out_shape=(jax.ShapeDtypeStruct((N_NODES, A_DIM), jnp.float32),
                   jax.ShapeDtypeStruct((N_NODES, 1), jnp.float32),
                   jax.ShapeDtypeStruct((N_NODES, D), jnp.float32)),
    )(y2, agg, dinv, b2, gamma, beta, Wa, ba, Wq, bq)


@jax.jit
def kernel(x, edge_index, W1, b1, W2, b2, gamma, beta, Wa, ba, Wq, bq):
    batch, seq, _ = x.shape
    x2d = x.reshape(batch * seq, D)

    e = edge_index.astype(jnp.int32)
    pad = EP - E_EDGES
    src = jnp.concatenate([e[0], jnp.zeros((pad,), jnp.int32)])
    dst = jnp.concatenate([e[1], jnp.full((pad,), DUMMY, jnp.int32)])
    src = src.reshape(NS, CPT, CHUNK)
    dst = dst.reshape(NS, CPT, CHUNK)

    zeros128 = jnp.zeros((CHUNK, D), jnp.float32)

    degp, rsrc, rdst, cnts = _route_kernel(src, dst)
    y1, dinv = _tc1(x2d, W1, degp)
    agg1 = _agg_kernel(y1, rsrc, rdst, cnts, zeros128)
    y2 = _tc2(y1, agg1, dinv, b1.reshape(1, D), W2)
    agg2 = _agg_kernel(y2, rsrc, rdst, cnts, zeros128)
    logits, sv, h = _tc3(y2, agg2, dinv, b2.reshape(1, D),
                         gamma.reshape(1, D), beta.reshape(1, D),
                         Wa, ba.reshape(1, A_DIM), Wq, bq.reshape(1, A_DIM))
    return (logits, sv, h)
